# baseline (device time: 19993 ns/iter reference)
import jax
import jax.numpy as jnp
from jax import lax
from jax.experimental import pallas as pl
from jax.experimental.pallas import tpu as pltpu

N_DEV = 4


def kernel(x):
    x = x.astype(jnp.bfloat16)
    m, n_total = x.shape
    blk = n_total // N_DEV
    out_rows = N_DEV * m

    def body(x_ref, out_ref, send_sems, recv_sems):
        me = lax.axis_index("i")

        barrier_sem = pltpu.get_barrier_semaphore()
        for o in range(1, N_DEV):
            pl.semaphore_signal(
                barrier_sem,
                inc=1,
                device_id=((me + o) % N_DEV,),
                device_id_type=pl.DeviceIdType.MESH,
            )
        pl.semaphore_wait(barrier_sem, N_DEV - 1)

        sends = []
        for o in range(1, N_DEV):
            t = (me + o) % N_DEV
            rdma = pltpu.make_async_remote_copy(
                src_ref=x_ref.at[:, pl.ds(t * blk, blk)],
                dst_ref=out_ref.at[pl.ds(me * m, m), :],
                send_sem=send_sems.at[o],
                recv_sem=recv_sems.at[o],
                device_id=(t,),
                device_id_type=pl.DeviceIdType.MESH,
            )
            rdma.start()
            sends.append(rdma)

        out_ref[pl.ds(me * m, m), :] = x_ref[:, pl.ds(me * blk, blk)]

        for o in range(1, N_DEV):
            s = (me - o) % N_DEV
            recv = pltpu.make_async_remote_copy(
                src_ref=x_ref.at[:, pl.ds(s * blk, blk)],
                dst_ref=out_ref.at[pl.ds(s * m, m), :],
                send_sem=send_sems.at[o],
                recv_sem=recv_sems.at[o],
                device_id=(s,),
                device_id_type=pl.DeviceIdType.MESH,
            )
            recv.wait_recv()

        for rdma in sends:
            rdma.wait_send()

    return pl.pallas_call(
        body,
        out_shape=jax.ShapeDtypeStruct((out_rows, blk), jnp.bfloat16),
        in_specs=[pl.BlockSpec(memory_space=pltpu.VMEM)],
        out_specs=pl.BlockSpec(memory_space=pltpu.VMEM),
        scratch_shapes=[
            pltpu.SemaphoreType.DMA((N_DEV,)),
            pltpu.SemaphoreType.DMA((N_DEV,)),
        ],
        compiler_params=pltpu.CompilerParams(collective_id=0),
    )(x)


# device time: 19983 ns/iter; 1.0005x vs baseline; 1.0005x over previous
import jax
import jax.numpy as jnp
from jax import lax
from jax.experimental import pallas as pl
from jax.experimental.pallas import tpu as pltpu

N_DEV = 4


def kernel(x):
    x = x.astype(jnp.bfloat16)
    m, n_total = x.shape
    blk = n_total // N_DEV
    out_rows = N_DEV * m

    def body(x_ref, out_ref, send_sems, recv_sems):
        me = lax.axis_index("i")

        barrier_sem = pltpu.get_barrier_semaphore()
        for o in range(1, N_DEV):
            pl.semaphore_signal(
                barrier_sem,
                inc=1,
                device_id=((me + o) % N_DEV,),
                device_id_type=pl.DeviceIdType.MESH,
            )
        pl.semaphore_wait(barrier_sem, N_DEV - 1)

        sends = []
        for o in (2, 1, 3):
            t = (me + o) % N_DEV
            rdma = pltpu.make_async_remote_copy(
                src_ref=x_ref.at[:, pl.ds(t * blk, blk)],
                dst_ref=out_ref.at[pl.ds(me * m, m), :],
                send_sem=send_sems.at[o],
                recv_sem=recv_sems.at[o],
                device_id=(t,),
                device_id_type=pl.DeviceIdType.MESH,
            )
            rdma.start()
            sends.append(rdma)

        out_ref[pl.ds(me * m, m), :] = x_ref[:, pl.ds(me * blk, blk)]

        for o in range(1, N_DEV):
            s = (me - o) % N_DEV
            recv = pltpu.make_async_remote_copy(
                src_ref=x_ref.at[:, pl.ds(s * blk, blk)],
                dst_ref=out_ref.at[pl.ds(s * m, m), :],
                send_sem=send_sems.at[o],
                recv_sem=recv_sems.at[o],
                device_id=(s,),
                device_id_type=pl.DeviceIdType.MESH,
            )
            recv.wait_recv()

        for rdma in sends:
            rdma.wait_send()

    return pl.pallas_call(
        body,
        out_shape=jax.ShapeDtypeStruct((out_rows, blk), jnp.bfloat16),
        in_specs=[pl.BlockSpec(memory_space=pltpu.VMEM)],
        out_specs=pl.BlockSpec(memory_space=pltpu.VMEM),
        scratch_shapes=[
            pltpu.SemaphoreType.DMA((N_DEV,)),
            pltpu.SemaphoreType.DMA((N_DEV,)),
        ],
        compiler_params=pltpu.CompilerParams(collective_id=0),
    )(x)
